# baseline (speedup 1.0000x reference)
import functools

import jax
import jax.numpy as jnp
from jax import lax
from jax.experimental import pallas as pl
from jax.experimental.pallas import tpu as pltpu
from jax.experimental.pallas import tpu_sc as plsc

B = 4096
T = 200
D = 64
VOCAB = 1000000

NC = 2
NS = 16
NW = NC * NS
ROWS = B * T
RPW = ROWS // NW
C = 1600
G = 16
GI = C // G
NCH = RPW // C
NCHUNKS = ROWS // C


def _sc_body(tokens_hbm, table_hbm, pos_hbm, out_hbm, idx_v, buf, pos_v, sem):
    wid = lax.axis_index("s") * NC + lax.axis_index("c")

    pltpu.sync_copy(pos_hbm, pos_v)

    def chunk_body(c, carry):
        cid = wid * NCH + c
        pltpu.sync_copy(tokens_hbm.at[cid], idx_v)
        copies = []
        for g in range(G):
            copies.append(
                pltpu.make_async_copy(
                    table_hbm.at[idx_v.at[g]],
                    buf.at[pl.ds(g * GI, GI)],
                    sem,
                )
            )
        for cp in copies:
            cp.start()
        for cp in copies:
            cp.wait()

        def pos_body(p, _):
            for q in range(D // 16):
                pv = pos_v[p, pl.ds(q * 16, 16)]
                for rep in range(C // T):
                    plsc.addupdate(buf.at[p + rep * T, pl.ds(q * 16, 16)], pv)
            return _

        lax.fori_loop(0, T, pos_body, 0)

        pltpu.sync_copy(buf, out_hbm.at[cid])
        return carry

    lax.fori_loop(0, NCH, chunk_body, 0)


@jax.jit
def _sc_lookup(tokens_r, emb_weight, pos):
    mesh = plsc.VectorSubcoreMesh(core_axis_name="c", subcore_axis_name="s")
    fn = pl.kernel(
        _sc_body,
        out_type=jax.ShapeDtypeStruct((NCHUNKS, C, D), jnp.float32),
        mesh=mesh,
        scratch_types=[
            pltpu.VMEM((G, GI), jnp.int32),
            pltpu.VMEM((C, D), jnp.float32),
            pltpu.VMEM((T, D), jnp.float32),
            pltpu.SemaphoreType.DMA,
        ],
        compiler_params=pltpu.CompilerParams(use_tc_tiling_on_sc=False),
    )
    return fn(tokens_r, emb_weight, pos)


def kernel(tokens, emb_weight, pos):
    tokens_r = tokens.astype(jnp.int32).reshape(NCHUNKS, G, GI)
    out = _sc_lookup(tokens_r, emb_weight, pos)
    return out.reshape(B, T, D)
